# baseline (speedup 1.0000x reference)
import functools

import jax
import jax.numpy as jnp
from jax import lax
from jax.experimental import pallas as pl
from jax.experimental.pallas import tpu as pltpu
from jax.experimental.pallas import tpu_sc as plsc

N = 10000
E = 160000
H = 192
NC = 2
NS = 16
C = 96
K = 105
EPW = C * K
EPAD = NS * EPW
ACC_ROWS = 10112
ROWS_PER_SUB = ACC_ROWS // NS
QW = 96
DEG_ROWS = 10240

_mesh = plsc.VectorSubcoreMesh(core_axis_name="c", subcore_axis_name="s")


def _gelu_approx(v):
    c1 = -1.5957691216057308
    c2 = -0.07135481627159393
    z2 = v * (c2 * (v * v) + c1)
    z2 = jnp.minimum(z2, 30.0)
    d = 1.0 + jnp.exp(z2)
    r = plsc.bitcast(jnp.full(v.shape, 0x7EF311C3, jnp.int32)
                     - plsc.bitcast(d, jnp.int32), jnp.float32)
    r = r * (2.0 - d * r)
    r = r * (2.0 - d * r)
    return v * r


def _sc_layer_body(tbl_s, tbl_d, srcs, dsts, eas, wb, out,
                   ix_s0, ix_s1, ix_d0, ix_d1, src_v, dst_v, ea_v,
                   bxs0, bxs1, bxd0, bxd1, wv, acc, sgs, sgd, ssc0, ssc1):
    c = lax.axis_index("c")
    s = lax.axis_index("s")

    pltpu.sync_copy(srcs.at[s], src_v)
    pltpu.sync_copy(dsts.at[s], dst_v)
    pltpu.sync_copy(eas.at[s], ea_v)
    z16 = jnp.zeros((16,), jnp.float32)
    NF = QW // 16

    ix_s = (ix_s0, ix_s1)
    ix_d = (ix_d0, ix_d1)
    bxs = (bxs0, bxs1)
    bxd = (bxd0, bxd1)

    for p in range(2):
        q = 2 * p + c
        qn16 = jnp.full((16,), q * N, jnp.int32)
        pltpu.sync_copy(wb.at[q], wv)

        def _zero_buf(i, _):
            for f in range(NF):
                bxs0[i, pl.ds(f * 16, 16)] = z16
            return 0
        lax.fori_loop(0, C, _zero_buf, 0)
        base = s * ROWS_PER_SUB
        for r0 in range(0, ROWS_PER_SUB, C):
            rows = min(C, ROWS_PER_SUB - r0)
            pltpu.sync_copy(bxs0.at[pl.ds(0, rows)],
                            acc.at[pl.ds(base + r0, rows)])
        plsc.subcore_barrier()

        w1e = [wv[0, pl.ds(f * 16, 16)] for f in range(NF)]
        b1 = [wv[1, pl.ds(f * 16, 16)] for f in range(NF)]

        def _build_rings(slot, j):
            for f in range(C // 16):
                ix_s[slot][pl.ds(f * 16, 16)] = (
                    src_v[j, pl.ds(f * 16, 16)] + qn16)
                ix_d[slot][pl.ds(f * 16, 16)] = jnp.minimum(
                    dst_v[j, pl.ds(f * 16, 16)] + qn16, 4 * N - 1)

        def _issue_gathers(slot):
            gs = pltpu.async_copy(tbl_s.at[ix_s[slot]], bxs[slot], sgs)
            gd = pltpu.async_copy(tbl_d.at[ix_d[slot]], bxd[slot], sgd)
            return gs, gd

        ssc = (ssc0, ssc1)

        def _compute_scatter(slot, j):
            j16 = jnp.full((16,), j, jnp.int32)
            bx, bd = bxs[slot], bxd[slot]

            def _edge(e, _):
                ea16 = plsc.load_gather(
                    ea_v, [j16, jnp.full((16,), e, jnp.int32)])
                xs = [bx[e, pl.ds(f * 16, 16)] for f in range(NF)]
                xd = [bd[e, pl.ds(f * 16, 16)] for f in range(NF)]
                g = [_gelu_approx(xs[f] + xd[f] + (ea16 * w1e[f] + b1[f]))
                     for f in range(NF)]
                for f in range(NF):
                    bx[e, pl.ds(f * 16, 16)] = g[f]
                return 0
            lax.fori_loop(0, C, _edge, 0)
            pltpu.async_copy(bx, acc.at[dst_v.at[j]], ssc[slot], add=True)

        def _wait_scatter(slot, j):
            pltpu.make_async_copy(bxs[slot], acc.at[dst_v.at[j]],
                                  ssc[slot]).wait()

        def _substep(slot, j):
            _build_rings(1 - slot, j + 1)
            _wait_scatter(1 - slot, j - 1)
            gs, gd = _issue_gathers(1 - slot)
            _compute_scatter(slot, j)
            gs.wait()
            gd.wait()

        _build_rings(0, jnp.int32(0))
        gs0, gd0 = _issue_gathers(0)
        gs0.wait()
        gd0.wait()
        _build_rings(1, jnp.int32(1))
        gs1, gd1 = _issue_gathers(1)
        _compute_scatter(0, jnp.int32(0))
        gs1.wait()
        gd1.wait()

        def _main(t, _):
            j = 2 * t + 1
            _substep(1, j)
            _substep(0, j + 1)
            return 0
        lax.fori_loop(0, (K - 3) // 2, _main, 0)
        _substep(1, jnp.int32(K - 2))
        _compute_scatter(0, jnp.int32(K - 1))
        _wait_scatter(1, jnp.int32(K - 2))
        _wait_scatter(0, jnp.int32(K - 1))

        plsc.subcore_barrier()
        pltpu.sync_copy(acc.at[pl.ds(s * ROWS_PER_SUB, ROWS_PER_SUB)],
                        out.at[p, c, pl.ds(s * ROWS_PER_SUB, ROWS_PER_SUB)])
        plsc.subcore_barrier()


_sc_layer = functools.partial(
    pl.kernel,
    out_type=jax.ShapeDtypeStruct((2, NC, ACC_ROWS, QW), jnp.float32),
    mesh=_mesh,
    compiler_params=pltpu.CompilerParams(needs_layout_passes=False,
                                         use_tc_tiling_on_sc=False),
    scratch_types=[
        pltpu.VMEM((C,), jnp.int32),
        pltpu.VMEM((C,), jnp.int32),
        pltpu.VMEM((C,), jnp.int32),
        pltpu.VMEM((C,), jnp.int32),
        pltpu.VMEM((K, C), jnp.int32),
        pltpu.VMEM((K, C), jnp.int32),
        pltpu.VMEM((K, C), jnp.float32),
        pltpu.VMEM((C, QW), jnp.float32),
        pltpu.VMEM((C, QW), jnp.float32),
        pltpu.VMEM((C, QW), jnp.float32),
        pltpu.VMEM((C, QW), jnp.float32),
        pltpu.VMEM((2, QW), jnp.float32),
        pltpu.VMEM_SHARED((ACC_ROWS, QW), jnp.float32),
        pltpu.SemaphoreType.DMA, pltpu.SemaphoreType.DMA,
        pltpu.SemaphoreType.DMA, pltpu.SemaphoreType.DMA,
    ],
)(_sc_layer_body)


def _sc_deg_body(dsts, out, dst_v, deg_v, red_v, sh, sem):
    c = lax.axis_index("c")
    s = lax.axis_index("s")

    @pl.when(c == 0)
    def _():
        pltpu.sync_copy(dsts.at[s], dst_v)

        def _zero(i, _):
            deg_v[pl.ds(i * 16, 16)] = jnp.zeros((16,), jnp.float32)
            return 0
        lax.fori_loop(0, DEG_ROWS // 16, _zero, 0)
        ones = jnp.ones((16,), jnp.float32)

        def _count(i, _):
            jj = i // (C // 16)
            ee = (i % (C // 16)) * 16
            idx = dst_v[jj, pl.ds(ee, 16)]
            plsc.addupdate_scatter(deg_v, [idx], ones)
            return 0
        lax.fori_loop(0, K * (C // 16), _count, 0)
        pltpu.sync_copy(deg_v, sh.at[s])
        plsc.subcore_barrier()
        col0 = s * (DEG_ROWS // NS)
        pltpu.sync_copy(sh.at[:, pl.ds(col0, DEG_ROWS // NS)], red_v)

        def _red(f, _):
            acc16 = red_v[0, pl.ds(f * 16, 16)]
            for t in range(1, NS):
                acc16 = acc16 + red_v[t, pl.ds(f * 16, 16)]
            deg_v[pl.ds(f * 16, 16)] = acc16
            return 0
        lax.fori_loop(0, DEG_ROWS // NS // 16, _red, 0)
        pltpu.sync_copy(deg_v.at[pl.ds(0, DEG_ROWS // NS)],
                        out.at[pl.ds(col0, DEG_ROWS // NS)])


_sc_deg = functools.partial(
    pl.kernel,
    out_type=jax.ShapeDtypeStruct((DEG_ROWS,), jnp.float32),
    mesh=_mesh,
    compiler_params=pltpu.CompilerParams(needs_layout_passes=False,
                                         use_tc_tiling_on_sc=False),
    scratch_types=[
        pltpu.VMEM((K, C), jnp.int32),
        pltpu.VMEM((DEG_ROWS,), jnp.float32),
        pltpu.VMEM((NS, DEG_ROWS // NS), jnp.float32),
        pltpu.VMEM_SHARED((NS, DEG_ROWS), jnp.float32),
        pltpu.SemaphoreType.DMA,
    ],
)(_sc_deg_body)


def _gelu_tc(x):
    return 0.5 * x * (1.0 + lax.erf(x * 0.7071067811865476))


def _ln(x, g, bt):
    mu = jnp.mean(x, axis=-1, keepdims=True)
    var = jnp.mean((x - mu) ** 2, axis=-1, keepdims=True)
    return (x - mu) * lax.rsqrt(var + 1e-5) * g + bt


def _dot(a, b):
    return jnp.dot(a, b, preferred_element_type=jnp.float32,
                   precision=lax.Precision.HIGHEST)


def _tables(x, w1r_ref):
    return jnp.stack(
        [jnp.stack([_dot(x, w1r_ref[t, q]) for q in range(4)])
         for t in range(2)])


def _enc_pre_body(af_ref, w_ref, b_ref, g_ref, bt_ref, w1r_ref,
                  x_ref, tbl_ref):
    x = _dot(af_ref[...], w_ref[...]) + b_ref[...]
    x = _ln(_gelu_tc(x), g_ref[...], bt_ref[...])
    x_ref[...] = x
    tbl_ref[0] = _tables(x, w1r_ref)


def _enc_pre(af, enc, w1):
    nb = 2000
    w1r = w1[:2 * H].reshape(2, H, 4, QW).transpose(0, 2, 1, 3)
    return pl.pallas_call(
        _enc_pre_body,
        grid=(N // nb,),
        in_specs=[
            pl.BlockSpec((nb, 4), lambda i: (i, 0)),
            pl.BlockSpec((4, H), lambda i: (0, 0)),
            pl.BlockSpec((1, H), lambda i: (0, 0)),
            pl.BlockSpec((1, H), lambda i: (0, 0)),
            pl.BlockSpec((1, H), lambda i: (0, 0)),
            pl.BlockSpec((2, 4, H, QW), lambda i: (0, 0, 0, 0)),
        ],
        out_specs=(
            pl.BlockSpec((nb, H), lambda i: (i, 0)),
            pl.BlockSpec((1, 2, 4, nb, QW), lambda i: (0, 0, 0, i, 0)),
        ),
        out_shape=(jax.ShapeDtypeStruct((N, H), jnp.float32),
                   jax.ShapeDtypeStruct((1, 2, 4, N, QW), jnp.float32)),
    )(af, enc['w'], enc['b'].reshape(1, H), enc['g'].reshape(1, H),
      enc['bt'].reshape(1, H), w1r)


def _agg_x(x_ref, s_ref, w2_ref, deg_ref, b2_ref, g_ref, bt_ref):
    agg = _dot(s_ref[0], w2_ref[0])
    for q in range(1, 4):
        agg = agg + _dot(s_ref[q], w2_ref[q])
    agg = agg + deg_ref[...] * b2_ref[...]
    return _ln(x_ref[...] + agg, g_ref[...], bt_ref[...])


def _post_pre_body(x_ref, s_ref, w2_ref, deg_ref, b2_ref, g_ref, bt_ref,
                   w1r_ref, xn_ref, tbl_ref):
    xn = _agg_x(x_ref, s_ref, w2_ref, deg_ref, b2_ref, g_ref, bt_ref)
    xn_ref[...] = xn
    tbl_ref[0] = _tables(xn, w1r_ref)


def _post_pre(x, S, lp, deg, w1_next):
    nb = 2000
    w1r = w1_next[:2 * H].reshape(2, H, 4, QW).transpose(0, 2, 1, 3)
    return pl.pallas_call(
        _post_pre_body,
        grid=(N // nb,),
        in_specs=[
            pl.BlockSpec((nb, H), lambda i: (i, 0)),
            pl.BlockSpec((4, nb, QW), lambda i: (0, i, 0)),
            pl.BlockSpec((4, QW, H), lambda i: (0, 0, 0)),
            pl.BlockSpec((nb, 1), lambda i: (i, 0)),
            pl.BlockSpec((1, H), lambda i: (0, 0)),
            pl.BlockSpec((1, H), lambda i: (0, 0)),
            pl.BlockSpec((1, H), lambda i: (0, 0)),
            pl.BlockSpec((2, 4, H, QW), lambda i: (0, 0, 0, 0)),
        ],
        out_specs=(
            pl.BlockSpec((nb, H), lambda i: (i, 0)),
            pl.BlockSpec((1, 2, 4, nb, QW), lambda i: (0, 0, 0, i, 0)),
        ),
        out_shape=(jax.ShapeDtypeStruct((N, H), jnp.float32),
                   jax.ShapeDtypeStruct((1, 2, 4, N, QW), jnp.float32)),
    )(x, S, lp['w2'].reshape(4, QW, H), deg, lp['b2'].reshape(1, H),
      lp['g'].reshape(1, H), lp['bt'].reshape(1, H), w1r)


def _post_body(x_ref, s_ref, w2_ref, deg_ref, b2_ref, g_ref, bt_ref,
               xn_ref):
    xn_ref[...] = _agg_x(x_ref, s_ref, w2_ref, deg_ref, b2_ref, g_ref,
                         bt_ref)


def _post(x, S, lp, deg):
    nb = 2000
    return pl.pallas_call(
        _post_body,
        grid=(N // nb,),
        in_specs=[
            pl.BlockSpec((nb, H), lambda i: (i, 0)),
            pl.BlockSpec((4, nb, QW), lambda i: (0, i, 0)),
            pl.BlockSpec((4, QW, H), lambda i: (0, 0, 0)),
            pl.BlockSpec((nb, 1), lambda i: (i, 0)),
            pl.BlockSpec((1, H), lambda i: (0, 0)),
            pl.BlockSpec((1, H), lambda i: (0, 0)),
            pl.BlockSpec((1, H), lambda i: (0, 0)),
        ],
        out_specs=pl.BlockSpec((nb, H), lambda i: (i, 0)),
        out_shape=jax.ShapeDtypeStruct((N, H), jnp.float32),
    )(x, S, lp['w2'].reshape(4, QW, H), deg, lp['b2'].reshape(1, H),
      lp['g'].reshape(1, H), lp['bt'].reshape(1, H))


def _head_body(x_ref, w1_ref, b1_ref, w2_ref, b2_ref, w3_ref, b3_ref,
               out_ref):
    x = x_ref[...]
    mean_pool = jnp.mean(x, axis=0, keepdims=True)
    max_pool = jnp.max(x, axis=0, keepdims=True)
    ge = jnp.concatenate([mean_pool, max_pool], axis=-1)
    h = _gelu_tc(_dot(ge, w1_ref[...]) + b1_ref[...])
    h = _gelu_tc(_dot(h, w2_ref[...]) + b2_ref[...])
    out_ref[...] = _dot(h, w3_ref[...]) + b3_ref[...]


def _head(x, hp):
    return pl.pallas_call(
        _head_body,
        out_shape=jax.ShapeDtypeStruct((1, 200), jnp.float32),
    )(x, hp['w1'], hp['b1'].reshape(1, -1), hp['w2'], hp['b2'].reshape(1, -1),
      hp['w3'], hp['b3'].reshape(1, -1))


def kernel(atom_features, edge_index, edge_attr, params):
    src = edge_index[0]
    dst = edge_index[1]
    pad = EPAD - E
    src_p = jnp.concatenate([src, jnp.zeros((pad,), jnp.int32)])
    dst_s = jnp.concatenate([dst, jnp.full((pad,), N, jnp.int32)])
    ea_p = jnp.concatenate([edge_attr[:, 0], jnp.zeros((pad,), jnp.float32)])

    srcs = src_p.reshape(NS, K, C)
    dsts = dst_s.reshape(NS, K, C)
    eas = ea_p.reshape(NS, K, C)

    deg_full = _sc_deg(dsts)
    deg = deg_full[:N].reshape(N, 1)

    layers = params['layers']
    x, tbl5 = _enc_pre(atom_features, params['enc'], layers[0]['w1'])
    tbl = tbl5[0]
    for li, lp in enumerate(layers):
        wb = jnp.stack([lp['w1'][2 * H].reshape(4, QW),
                        lp['b1'].reshape(4, QW)], axis=1)
        S = _sc_layer(tbl[0].reshape(4 * N, QW), tbl[1].reshape(4 * N, QW),
                      srcs, dsts, eas, wb)
        S4 = S.reshape(4, ACC_ROWS, QW)
        if li + 1 < len(layers):
            x, tbl5 = _post_pre(x, S4[:, :N], lp, deg, layers[li + 1]['w1'])
            tbl = tbl5[0]
        else:
            x = _post(x, S4[:, :N], lp, deg)
            return _head(x, params['head'])


# speedup vs baseline: 1.0480x; 1.0480x over previous
import functools

import jax
import jax.numpy as jnp
from jax import lax
from jax.experimental import pallas as pl
from jax.experimental.pallas import tpu as pltpu
from jax.experimental.pallas import tpu_sc as plsc

N = 10000
E = 160000
H = 192
NC = 2
NS = 16
C = 96
K = 105
EPW = C * K
EPAD = NS * EPW
ACC_ROWS = 10112
ROWS_PER_SUB = ACC_ROWS // NS
QW = 96
DEG_ROWS = 10240

_mesh = plsc.VectorSubcoreMesh(core_axis_name="c", subcore_axis_name="s")


def _gelu_approx(v):
    c1 = -1.5957691216057308
    c2 = -0.07135481627159393
    z2 = v * (c2 * (v * v) + c1)
    z2 = jnp.minimum(z2, 30.0)
    d = 1.0 + jnp.exp(z2)
    r = plsc.bitcast(jnp.full(v.shape, 0x7EF311C3, jnp.int32)
                     - plsc.bitcast(d, jnp.int32), jnp.float32)
    r = r * (2.0 - d * r)
    r = r * (2.0 - d * r)
    return v * r


def _sc_layer_body(tbl_s, tbl_d, srcs, dsts, eas, wb, out,
                   ix_s0, ix_s1, ix_d0, ix_d1, src_v, dst_v, ea_v,
                   bxs0, bxs1, bxd0, bxd1, wv, acc, sgs, sgd, ssc0, ssc1):
    c = lax.axis_index("c")
    s = lax.axis_index("s")

    pltpu.sync_copy(srcs.at[s], src_v)
    pltpu.sync_copy(dsts.at[s], dst_v)
    pltpu.sync_copy(eas.at[s], ea_v)
    z16 = jnp.zeros((16,), jnp.float32)
    NF = QW // 16

    ix_s = (ix_s0, ix_s1)
    ix_d = (ix_d0, ix_d1)
    bxs = (bxs0, bxs1)
    bxd = (bxd0, bxd1)

    for p in range(2):
        q = 2 * p + c
        qn16 = jnp.full((16,), q * N, jnp.int32)
        pltpu.sync_copy(wb.at[q], wv)

        def _zero_buf(i, _):
            for f in range(NF):
                bxs0[i, pl.ds(f * 16, 16)] = z16
            return 0
        lax.fori_loop(0, C, _zero_buf, 0)
        base = s * ROWS_PER_SUB
        for r0 in range(0, ROWS_PER_SUB, C):
            rows = min(C, ROWS_PER_SUB - r0)
            pltpu.sync_copy(bxs0.at[pl.ds(0, rows)],
                            acc.at[pl.ds(base + r0, rows)])
        plsc.subcore_barrier()

        w1e = [wv[0, pl.ds(f * 16, 16)] for f in range(NF)]
        b1 = [wv[1, pl.ds(f * 16, 16)] for f in range(NF)]

        def _build_rings(slot, j):
            for f in range(C // 16):
                ix_s[slot][pl.ds(f * 16, 16)] = (
                    src_v[j, pl.ds(f * 16, 16)] + qn16)
                ix_d[slot][pl.ds(f * 16, 16)] = jnp.minimum(
                    dst_v[j, pl.ds(f * 16, 16)] + qn16, 4 * N - 1)

        def _issue_gathers(slot):
            gs = pltpu.async_copy(tbl_s.at[ix_s[slot]], bxs[slot], sgs)
            gd = pltpu.async_copy(tbl_d.at[ix_d[slot]], bxd[slot], sgd)
            return gs, gd

        ssc = (ssc0, ssc1)

        def _compute_scatter(slot, j):
            j16 = jnp.full((16,), j, jnp.int32)
            bx, bd = bxs[slot], bxd[slot]

            def _edge(e, _):
                ea16 = plsc.load_gather(
                    ea_v, [j16, jnp.full((16,), e, jnp.int32)])
                xs = [bx[e, pl.ds(f * 16, 16)] for f in range(NF)]
                xd = [bd[e, pl.ds(f * 16, 16)] for f in range(NF)]
                g = [_gelu_approx(xs[f] + xd[f] + (ea16 * w1e[f] + b1[f]))
                     for f in range(NF)]
                for f in range(NF):
                    bx[e, pl.ds(f * 16, 16)] = g[f]
                return 0
            lax.fori_loop(0, C, _edge, 0)
            pltpu.async_copy(bx, acc.at[dst_v.at[j]], ssc[slot], add=True)

        def _wait_scatter(slot, j):
            pltpu.make_async_copy(bxs[slot], acc.at[dst_v.at[j]],
                                  ssc[slot]).wait()

        def _substep(slot, j):
            _build_rings(1 - slot, j + 1)
            _wait_scatter(1 - slot, j - 1)
            gs, gd = _issue_gathers(1 - slot)
            _compute_scatter(slot, j)
            gs.wait()
            gd.wait()

        _build_rings(0, jnp.int32(0))
        gs0, gd0 = _issue_gathers(0)
        gs0.wait()
        gd0.wait()
        _build_rings(1, jnp.int32(1))
        gs1, gd1 = _issue_gathers(1)
        _compute_scatter(0, jnp.int32(0))
        gs1.wait()
        gd1.wait()

        def _main(t, _):
            j = 2 * t + 1
            _substep(1, j)
            _substep(0, j + 1)
            return 0
        lax.fori_loop(0, (K - 3) // 2, _main, 0)
        _substep(1, jnp.int32(K - 2))
        _compute_scatter(0, jnp.int32(K - 1))
        _wait_scatter(1, jnp.int32(K - 2))
        _wait_scatter(0, jnp.int32(K - 1))

        plsc.subcore_barrier()
        pltpu.sync_copy(acc.at[pl.ds(s * ROWS_PER_SUB, ROWS_PER_SUB)],
                        out.at[p, c, pl.ds(s * ROWS_PER_SUB, ROWS_PER_SUB)])
        plsc.subcore_barrier()


_sc_layer = functools.partial(
    pl.kernel,
    out_type=jax.ShapeDtypeStruct((2, NC, ACC_ROWS, QW), jnp.float32),
    mesh=_mesh,
    compiler_params=pltpu.CompilerParams(needs_layout_passes=False,
                                         use_tc_tiling_on_sc=False),
    scratch_types=[
        pltpu.VMEM((C,), jnp.int32),
        pltpu.VMEM((C,), jnp.int32),
        pltpu.VMEM((C,), jnp.int32),
        pltpu.VMEM((C,), jnp.int32),
        pltpu.VMEM((K, C), jnp.int32),
        pltpu.VMEM((K, C), jnp.int32),
        pltpu.VMEM((K, C), jnp.float32),
        pltpu.VMEM((C, QW), jnp.float32),
        pltpu.VMEM((C, QW), jnp.float32),
        pltpu.VMEM((C, QW), jnp.float32),
        pltpu.VMEM((C, QW), jnp.float32),
        pltpu.VMEM((2, QW), jnp.float32),
        pltpu.VMEM_SHARED((ACC_ROWS, QW), jnp.float32),
        pltpu.SemaphoreType.DMA, pltpu.SemaphoreType.DMA,
        pltpu.SemaphoreType.DMA, pltpu.SemaphoreType.DMA,
    ],
)(_sc_layer_body)


def _sc_deg_body(dsts, out, dst_v, deg_v, red_v, sh, sem):
    c = lax.axis_index("c")
    s = lax.axis_index("s")

    @pl.when(c == 0)
    def _():
        pltpu.sync_copy(dsts.at[s], dst_v)

        def _zero(i, _):
            deg_v[pl.ds(i * 16, 16)] = jnp.zeros((16,), jnp.float32)
            return 0
        lax.fori_loop(0, DEG_ROWS // 16, _zero, 0)
        ones = jnp.ones((16,), jnp.float32)

        def _count(i, _):
            jj = i // (C // 16)
            ee = (i % (C // 16)) * 16
            idx = dst_v[jj, pl.ds(ee, 16)]
            plsc.addupdate_scatter(deg_v, [idx], ones)
            return 0
        lax.fori_loop(0, K * (C // 16), _count, 0)
        pltpu.sync_copy(deg_v, sh.at[s])
        plsc.subcore_barrier()
        col0 = s * (DEG_ROWS // NS)
        pltpu.sync_copy(sh.at[:, pl.ds(col0, DEG_ROWS // NS)], red_v)

        def _red(f, _):
            acc16 = red_v[0, pl.ds(f * 16, 16)]
            for t in range(1, NS):
                acc16 = acc16 + red_v[t, pl.ds(f * 16, 16)]
            deg_v[pl.ds(f * 16, 16)] = acc16
            return 0
        lax.fori_loop(0, DEG_ROWS // NS // 16, _red, 0)
        pltpu.sync_copy(deg_v.at[pl.ds(0, DEG_ROWS // NS)],
                        out.at[pl.ds(col0, DEG_ROWS // NS)])


_sc_deg = functools.partial(
    pl.kernel,
    out_type=jax.ShapeDtypeStruct((DEG_ROWS,), jnp.float32),
    mesh=_mesh,
    compiler_params=pltpu.CompilerParams(needs_layout_passes=False,
                                         use_tc_tiling_on_sc=False),
    scratch_types=[
        pltpu.VMEM((K, C), jnp.int32),
        pltpu.VMEM((DEG_ROWS,), jnp.float32),
        pltpu.VMEM((NS, DEG_ROWS // NS), jnp.float32),
        pltpu.VMEM_SHARED((NS, DEG_ROWS), jnp.float32),
        pltpu.SemaphoreType.DMA,
    ],
)(_sc_deg_body)


def _gelu_tc(x):
    return 0.5 * x * (1.0 + lax.erf(x * 0.7071067811865476))


def _enc_body(af_ref, w_ref, b_ref, g_ref, bt_ref, out_ref):
    x = jnp.dot(af_ref[...], w_ref[...], preferred_element_type=jnp.float32,
                 precision=lax.Precision.HIGHEST)
    x = x + b_ref[...]
    x = _gelu_tc(x)
    mu = jnp.mean(x, axis=-1, keepdims=True)
    var = jnp.mean((x - mu) ** 2, axis=-1, keepdims=True)
    out_ref[...] = (x - mu) * lax.rsqrt(var + 1e-5) * g_ref[...] + bt_ref[...]


def _encoder(af, w, b, g, bt):
    return pl.pallas_call(
        _enc_body,
        out_shape=jax.ShapeDtypeStruct((N, H), jnp.float32),
    )(af, w, b.reshape(1, H), g.reshape(1, H), bt.reshape(1, H))


def _pre_body(x_ref, w1_ref, out_ref):
    out_ref[0] = jnp.dot(x_ref[...], w1_ref[0, 0],
                         preferred_element_type=jnp.float32,
                 precision=lax.Precision.HIGHEST)


def _pre_tables(x, w1):
    w1r = w1[:2 * H].reshape(2, H, 4, QW).transpose(0, 2, 1, 3)
    return pl.pallas_call(
        _pre_body,
        grid=(2, 4),
        in_specs=[
            pl.BlockSpec((N, H), lambda t, q: (0, 0)),
            pl.BlockSpec((1, 1, H, QW), lambda t, q: (t, q, 0, 0)),
        ],
        out_specs=pl.BlockSpec((1, N, QW), lambda t, q: (t, q, 0)),
        out_shape=jax.ShapeDtypeStruct((2, 4 * N, QW), jnp.float32),
    )(x, w1r)


def _post_body(x_ref, s_ref, w2_ref, deg_ref, b2_ref, g_ref, bt_ref,
               out_ref):
    agg = jnp.dot(s_ref[0], w2_ref[0], preferred_element_type=jnp.float32,
                 precision=lax.Precision.HIGHEST)
    for q in range(1, 4):
        agg = agg + jnp.dot(s_ref[q], w2_ref[q],
                            preferred_element_type=jnp.float32,
                 precision=lax.Precision.HIGHEST)
    agg = agg + deg_ref[...] * b2_ref[...]
    x = x_ref[...] + agg
    mu = jnp.mean(x, axis=-1, keepdims=True)
    var = jnp.mean((x - mu) ** 2, axis=-1, keepdims=True)
    out_ref[...] = (x - mu) * lax.rsqrt(var + 1e-5) * g_ref[...] + bt_ref[...]


def _post(x, S, w2, deg, b2, g, bt):
    nb = 2000
    return pl.pallas_call(
        _post_body,
        grid=(N // nb,),
        in_specs=[
            pl.BlockSpec((nb, H), lambda i: (i, 0)),
            pl.BlockSpec((4, nb, QW), lambda i: (0, i, 0)),
            pl.BlockSpec((4, QW, H), lambda i: (0, 0, 0)),
            pl.BlockSpec((nb, 1), lambda i: (i, 0)),
            pl.BlockSpec((1, H), lambda i: (0, 0)),
            pl.BlockSpec((1, H), lambda i: (0, 0)),
            pl.BlockSpec((1, H), lambda i: (0, 0)),
        ],
        out_specs=pl.BlockSpec((nb, H), lambda i: (i, 0)),
        out_shape=jax.ShapeDtypeStruct((N, H), jnp.float32),
    )(x, S, w2.reshape(4, QW, H), deg, b2.reshape(1, H), g.reshape(1, H),
      bt.reshape(1, H))


def _head_body(x_ref, w1_ref, b1_ref, w2_ref, b2_ref, w3_ref, b3_ref,
               out_ref):
    x = x_ref[...]
    mean_pool = jnp.mean(x, axis=0, keepdims=True)
    max_pool = jnp.max(x, axis=0, keepdims=True)
    ge = jnp.concatenate([mean_pool, max_pool], axis=-1)
    h = _gelu_tc(jnp.dot(ge, w1_ref[...],
                         preferred_element_type=jnp.float32,
                 precision=lax.Precision.HIGHEST) + b1_ref[...])
    h = _gelu_tc(jnp.dot(h, w2_ref[...],
                         preferred_element_type=jnp.float32,
                 precision=lax.Precision.HIGHEST) + b2_ref[...])
    out_ref[...] = jnp.dot(h, w3_ref[...],
                           preferred_element_type=jnp.float32,
                 precision=lax.Precision.HIGHEST) + b3_ref[...]


def _head(x, hp):
    return pl.pallas_call(
        _head_body,
        out_shape=jax.ShapeDtypeStruct((1, 200), jnp.float32),
    )(x, hp['w1'], hp['b1'].reshape(1, -1), hp['w2'], hp['b2'].reshape(1, -1),
      hp['w3'], hp['b3'].reshape(1, -1))


def kernel(atom_features, edge_index, edge_attr, params):
    src = edge_index[0]
    dst = edge_index[1]
    pad = EPAD - E
    src_p = jnp.concatenate([src, jnp.zeros((pad,), jnp.int32)])
    dst_s = jnp.concatenate([dst, jnp.full((pad,), N, jnp.int32)])
    ea_p = jnp.concatenate([edge_attr[:, 0], jnp.zeros((pad,), jnp.float32)])

    srcs = src_p.reshape(NS, K, C)
    dsts = dst_s.reshape(NS, K, C)
    eas = ea_p.reshape(NS, K, C)

    deg_full = _sc_deg(dsts)
    deg = deg_full[:N].reshape(N, 1)

    enc = params['enc']
    x = _encoder(atom_features, enc['w'], enc['b'], enc['g'], enc['bt'])

    for lp in params['layers']:
        tbl = _pre_tables(x, lp['w1'])
        wb = jnp.stack([lp['w1'][2 * H].reshape(4, QW),
                        lp['b1'].reshape(4, QW)], axis=1)
        S = _sc_layer(tbl[0], tbl[1], srcs, dsts, eas, wb)
        S4 = S.reshape(4, ACC_ROWS, QW)
        x = _post(x, S4, lp['w2'], deg, lp['b2'], lp['g'], lp['bt'])

    return _head(x, params['head'])
